# baseline (speedup 1.0000x reference)
import jax
import jax.numpy as jnp
from jax.experimental import pallas as pl
from jax.experimental.pallas import tpu as pltpu

_NB = 8
_CHUNK = 32768


def _hist_kernel(x_ref, c_ref, w_ref, oned_ref, twod_ref):
    j = pl.program_id(1)
    nj = pl.num_programs(1)

    x = x_ref[0, 0, :]
    c = c_ref[0, :]
    w = w_ref[0, 0]

    z = w - jnp.abs(x[None, :] - c[:, None])
    p = jnp.power(jnp.float32(1.01), z)
    xx = jnp.where(p > 1.0, p, jnp.float32(0.0))

    twod_ref[0, :, :] = xx

    part = jnp.sum(xx, axis=1, keepdims=True)[None]

    @pl.when(j == 0)
    def _init():
        oned_ref[...] = jnp.zeros_like(oned_ref)

    oned_ref[...] += part

    @pl.when(j == nj - 1)
    def _finish():
        oned_ref[...] *= jnp.float32(1.0) / jnp.float32(x_ref.shape[-1] * nj)


def kernel(input_image, centers, width):
    B, C, H, W = input_image.shape
    NB = centers.shape[0]
    HW = H * W
    BC = B * C
    nj = HW // _CHUNK

    x3 = input_image.reshape(BC, 1, HW)
    c2 = centers.reshape(1, NB).astype(jnp.float32)
    w2 = jnp.asarray(width, jnp.float32).reshape(1, 1)

    oned, twod = pl.pallas_call(
        _hist_kernel,
        grid=(BC, nj),
        in_specs=[
            pl.BlockSpec((1, 1, _CHUNK), lambda i, j: (i, 0, j)),
            pl.BlockSpec((1, NB), lambda i, j: (0, 0)),
            pl.BlockSpec((1, 1), lambda i, j: (0, 0)),
        ],
        out_specs=[
            pl.BlockSpec((1, NB, 1), lambda i, j: (i, 0, 0)),
            pl.BlockSpec((1, NB, _CHUNK), lambda i, j: (i, 0, j)),
        ],
        out_shape=[
            jax.ShapeDtypeStruct((BC, NB, 1), jnp.float32),
            jax.ShapeDtypeStruct((BC, NB, HW), jnp.float32),
        ],
        compiler_params=pltpu.CompilerParams(
            dimension_semantics=("parallel", "arbitrary"),
        ),
    )(x3, c2, w2)

    one_d = oned.reshape(B, C * NB)
    two_d = twod.reshape(B, C * NB, HW)
    return one_d, two_d


# speedup vs baseline: 1.9143x; 1.9143x over previous
import jax
import jax.numpy as jnp
from jax.experimental import pallas as pl
from jax.experimental.pallas import tpu as pltpu

_NB = 8
_ROWS = 8
_CHUNK = 16384


def _hist_kernel(x_ref, c_ref, w_ref, oned_ref, twod_ref):
    j = pl.program_id(1)
    nj = pl.num_programs(1)

    x = x_ref[...]
    c = c_ref[0, :].reshape(_NB, 1)
    w = w_ref[0, 0]

    @pl.when(j == 0)
    def _init():
        oned_ref[...] = jnp.zeros_like(oned_ref)

    for k in range(_ROWS):
        xk = x[k:k + 1, :]
        z = w - jnp.abs(xk - c)
        p = jnp.power(jnp.float32(1.01), z)
        xx = jnp.where(p > 1.0, p, jnp.float32(0.0))
        twod_ref[k, :, :] = xx
        oned_ref[k, :, :] += jnp.sum(xx, axis=1, keepdims=True)

    @pl.when(j == nj - 1)
    def _finish():
        oned_ref[...] *= jnp.float32(1.0) / jnp.float32(_CHUNK * nj)


def kernel(input_image, centers, width):
    B, C, H, W = input_image.shape
    NB = centers.shape[0]
    HW = H * W
    BC = B * C
    nj = HW // _CHUNK
    ni = BC // _ROWS

    x2 = input_image.reshape(BC, HW)
    c2 = centers.reshape(1, NB).astype(jnp.float32)
    w2 = jnp.asarray(width, jnp.float32).reshape(1, 1)

    oned, twod = pl.pallas_call(
        _hist_kernel,
        grid=(ni, nj),
        in_specs=[
            pl.BlockSpec((_ROWS, _CHUNK), lambda i, j: (i, j)),
            pl.BlockSpec((1, NB), lambda i, j: (0, 0)),
            pl.BlockSpec((1, 1), lambda i, j: (0, 0)),
        ],
        out_specs=[
            pl.BlockSpec((_ROWS, NB, 1), lambda i, j: (i, 0, 0)),
            pl.BlockSpec((_ROWS, NB, _CHUNK), lambda i, j: (i, 0, j)),
        ],
        out_shape=[
            jax.ShapeDtypeStruct((BC, NB, 1), jnp.float32),
            jax.ShapeDtypeStruct((BC, NB, HW), jnp.float32),
        ],
        compiler_params=pltpu.CompilerParams(
            dimension_semantics=("parallel", "arbitrary"),
        ),
    )(x2, c2, w2)

    one_d = oned.reshape(B, C * NB)
    two_d = twod.reshape(B, C * NB, HW)
    return one_d, two_d
